# baseline (speedup 1.0000x reference)
import jax
import jax.numpy as jnp
from jax.experimental import pallas as pl
from jax.experimental.pallas import tpu as pltpu

_TEMPERATURE = 64.0
_Q_BLOCK = 240


def _protonet_body(s_hbm, q_ref, o_ref, s_vmem, proto_ref, sem):
    @pl.when(pl.program_id(0) == 0)
    def _():
        cp = pltpu.make_async_copy(s_hbm, s_vmem, sem)
        cp.start()
        cp.wait()
        proto_ref[...] = jnp.sum(s_vmem[...], axis=0) * (1.0 / s_vmem.shape[0])

    proto = proto_ref[...]
    q = q_ref[...]
    qn = jnp.sum(q * q, axis=1, keepdims=True)
    pn = jnp.sum(proto * proto, axis=1)[None, :]
    cross = jax.lax.dot_general(
        q, proto, (((1,), (1,)), ((), ())),
        preferred_element_type=jnp.float32,
    )
    o_ref[...] = (2.0 * cross - qn - pn) * (1.0 / _TEMPERATURE)


def kernel(support, query):
    n_batch, n_shot, n_way, emb_dim = support.shape
    n_query = n_batch * query.shape[1] * n_way
    s = support.reshape(n_shot, n_way, emb_dim)
    q = query.reshape(n_query, emb_dim)
    return pl.pallas_call(
        _protonet_body,
        grid=(n_query // _Q_BLOCK,),
        in_specs=[
            pl.BlockSpec(memory_space=pl.ANY),
            pl.BlockSpec((_Q_BLOCK, emb_dim), lambda i: (i, 0)),
        ],
        out_specs=pl.BlockSpec((_Q_BLOCK, n_way), lambda i: (i, 0)),
        out_shape=jax.ShapeDtypeStruct((n_query, n_way), jnp.float32),
        scratch_shapes=[
            pltpu.VMEM((n_shot, n_way, emb_dim), jnp.float32),
            pltpu.VMEM((n_way, emb_dim), jnp.float32),
            pltpu.SemaphoreType.DMA,
        ],
    )(s, q)


# speedup vs baseline: 1.3542x; 1.3542x over previous
import jax
import jax.numpy as jnp
from jax.experimental import pallas as pl
from jax.experimental.pallas import tpu as pltpu

_TEMPERATURE = 64.0


def _protonet_body(s_hbm, q_hbm, o_ref, s_vmem, q_vmem, sem_s, sem_q0, sem_q1):
    n_q = q_vmem.shape[0]
    half = n_q // 2
    cp_s = pltpu.make_async_copy(s_hbm, s_vmem, sem_s)
    cp_q0 = pltpu.make_async_copy(
        q_hbm.at[pl.ds(0, half)], q_vmem.at[pl.ds(0, half)], sem_q0)
    cp_q1 = pltpu.make_async_copy(
        q_hbm.at[pl.ds(half, half)], q_vmem.at[pl.ds(half, half)], sem_q1)
    cp_s.start()
    cp_q0.start()
    cp_q1.start()
    cp_s.wait()
    proto = jnp.sum(s_vmem[...], axis=0) * (1.0 / s_vmem.shape[0])
    pn = jnp.sum(proto * proto, axis=1)[None, :]

    def halve(cp, lo):
        cp.wait()
        q = q_vmem[pl.ds(lo, half), :]
        qn = jnp.sum(q * q, axis=1, keepdims=True)
        cross = jax.lax.dot_general(
            q, proto, (((1,), (1,)), ((), ())),
            preferred_element_type=jnp.float32,
        )
        o_ref[pl.ds(lo, half), :] = (2.0 * cross - qn - pn) * (1.0 / _TEMPERATURE)

    halve(cp_q0, 0)
    halve(cp_q1, half)


def kernel(support, query):
    n_batch, n_shot, n_way, emb_dim = support.shape
    n_query = n_batch * query.shape[1] * n_way
    s = support.reshape(n_shot, n_way, emb_dim)
    q = query.reshape(n_query, emb_dim)
    return pl.pallas_call(
        _protonet_body,
        in_specs=[
            pl.BlockSpec(memory_space=pl.ANY),
            pl.BlockSpec(memory_space=pl.ANY),
        ],
        out_shape=jax.ShapeDtypeStruct((n_query, n_way), jnp.float32),
        scratch_shapes=[
            pltpu.VMEM((n_shot, n_way, emb_dim), jnp.float32),
            pltpu.VMEM((n_query, emb_dim), jnp.float32),
            pltpu.SemaphoreType.DMA,
            pltpu.SemaphoreType.DMA,
            pltpu.SemaphoreType.DMA,
        ],
    )(s, q)
